# Initial kernel scaffold; baseline (speedup 1.0000x reference)
import jax
import jax.numpy as jnp
from jax.experimental import pallas as pl


def kernel(z, codebook):
    raise NotImplementedError("write your pallas kernel here")


# speedup vs baseline: 1.0915x; 1.0915x over previous
import functools

import jax
import jax.numpy as jnp
from jax.experimental import pallas as pl

_PATCH_DIM = 64
_K = 1024
_M_BLK = 2048


def _vq_body(z_ref, cb_ref, out_ref):
    p = z_ref[...]
    cb = cb_ref[...]
    x_sq = jnp.sum(p * p, axis=1, keepdims=True)
    c_sq = jnp.sum(cb * cb, axis=1)
    mm = jax.lax.dot_general(
        p, cb, (((1,), (1,)), ((), ())),
        preferred_element_type=jnp.float32)
    dist = x_sq + c_sq[None, :] - 2.0 * mm
    out_ref[0, 0, :] = jnp.argmin(dist, axis=1).astype(jnp.int32)


@functools.partial(jax.jit, static_argnames=())
def kernel(z, codebook):
    B, L = z.shape
    D = _PATCH_DIM
    M = B * (L // D)
    patches = z.reshape(M, D)
    grid = M // _M_BLK
    tokens = pl.pallas_call(
        _vq_body,
        grid=(grid,),
        in_specs=[
            pl.BlockSpec((_M_BLK, D), lambda i: (i, 0)),
            pl.BlockSpec((_K, D), lambda i: (0, 0)),
        ],
        out_specs=pl.BlockSpec((1, 1, _M_BLK), lambda i: (i, 0, 0)),
        out_shape=jax.ShapeDtypeStruct((grid, 1, _M_BLK), jnp.int32),
    )(patches, codebook)
    return tokens.reshape(B, L // D)


# speedup vs baseline: 1.2694x; 1.1630x over previous
import functools

import jax
import jax.numpy as jnp
from jax.experimental import pallas as pl

_PATCH_DIM = 64
_K = 1024
_M_BLK = 2048


def _vq_body(z_ref, cb_ref, out_ref):
    p = z_ref[...]
    cb = cb_ref[...]
    c_sq = jnp.sum(cb * cb, axis=1)
    x_sq = jnp.sum(p * p, axis=1, keepdims=True)
    mm2 = jax.lax.dot_general(
        p, cb * -2.0, (((1,), (1,)), ((), ())),
        preferred_element_type=jnp.float32)
    dist = (x_sq + c_sq[None, :]) + mm2
    mval = jnp.min(dist, axis=1, keepdims=True)
    iota = jax.lax.broadcasted_iota(jnp.int32, dist.shape, 1)
    kidx = jnp.min(jnp.where(dist == mval, iota, _K), axis=1)
    out_ref[0, 0, :] = kidx


@functools.partial(jax.jit, static_argnames=())
def kernel(z, codebook):
    B, L = z.shape
    D = _PATCH_DIM
    M = B * (L // D)
    patches = z.reshape(M, D)
    grid = M // _M_BLK
    tokens = pl.pallas_call(
        _vq_body,
        grid=(grid,),
        in_specs=[
            pl.BlockSpec((_M_BLK, D), lambda i: (i, 0)),
            pl.BlockSpec((_K, D), lambda i: (0, 0)),
        ],
        out_specs=pl.BlockSpec((1, 1, _M_BLK), lambda i: (i, 0, 0)),
        out_shape=jax.ShapeDtypeStruct((grid, 1, _M_BLK), jnp.int32),
    )(patches, codebook)
    return tokens.reshape(B, L // D)


# speedup vs baseline: 1.3210x; 1.0406x over previous
import functools

import jax
import jax.numpy as jnp
from jax.experimental import pallas as pl

_PATCH_DIM = 64
_K = 1024
_M_BLK = 2048


def _vq_body(z_ref, cb_ref, out_ref):
    p = z_ref[...]
    cb = cb_ref[...]
    c_sq = jnp.sum(cb * cb, axis=1)
    x_sq = jnp.sum(p * p, axis=1, keepdims=True)
    mm2 = jax.lax.dot_general(
        p, cb * -2.0, (((1,), (1,)), ((), ())),
        preferred_element_type=jnp.float32)
    dist = (x_sq + c_sq[None, :]) + mm2
    mval = jnp.min(dist, axis=1, keepdims=True)
    n_chunks = _K // 128
    lane = jax.lax.broadcasted_iota(jnp.int32, (p.shape[0], 128), 1)
    c = jnp.full((p.shape[0], 128), 2 * _K, jnp.int32)
    for i in reversed(range(n_chunks)):
        c = jnp.where(dist[:, i * 128:(i + 1) * 128] == mval, i * 128 + lane, c)
    out_ref[0, 0, :] = jnp.min(c, axis=1)


@functools.partial(jax.jit, static_argnames=())
def kernel(z, codebook):
    B, L = z.shape
    D = _PATCH_DIM
    M = B * (L // D)
    patches = z.reshape(M, D)
    grid = M // _M_BLK
    tokens = pl.pallas_call(
        _vq_body,
        grid=(grid,),
        in_specs=[
            pl.BlockSpec((_M_BLK, D), lambda i: (i, 0)),
            pl.BlockSpec((_K, D), lambda i: (0, 0)),
        ],
        out_specs=pl.BlockSpec((1, 1, _M_BLK), lambda i: (i, 0, 0)),
        out_shape=jax.ShapeDtypeStruct((grid, 1, _M_BLK), jnp.int32),
    )(patches, codebook)
    return tokens.reshape(B, L // D)


# speedup vs baseline: 1.4833x; 1.1229x over previous
import functools

import jax
import jax.numpy as jnp
from jax.experimental import pallas as pl

_PATCH_DIM = 64
_K = 1024
_M_BLK = 8192


def _vq_body(z_ref, cb_ref, out_ref):
    p = z_ref[...]
    cb = cb_ref[...]
    c_sq = jnp.sum(cb * cb, axis=1)
    x_sq = jnp.sum(p * p, axis=1, keepdims=True)
    mm2 = jax.lax.dot_general(
        p, cb * -2.0, (((1,), (1,)), ((), ())),
        preferred_element_type=jnp.float32)
    dist = (x_sq + c_sq[None, :]) + mm2
    mval = jnp.min(dist, axis=1, keepdims=True)
    n_chunks = _K // 128
    lane = jax.lax.broadcasted_iota(jnp.int32, (p.shape[0], 128), 1)
    c = jnp.full((p.shape[0], 128), 2 * _K, jnp.int32)
    for i in reversed(range(n_chunks)):
        c = jnp.where(dist[:, i * 128:(i + 1) * 128] == mval, i * 128 + lane, c)
    out_ref[0, 0, :] = jnp.min(c, axis=1)


@functools.partial(jax.jit, static_argnames=())
def kernel(z, codebook):
    B, L = z.shape
    D = _PATCH_DIM
    M = B * (L // D)
    patches = z.reshape(M, D)
    grid = M // _M_BLK
    tokens = pl.pallas_call(
        _vq_body,
        grid=(grid,),
        in_specs=[
            pl.BlockSpec((_M_BLK, D), lambda i: (i, 0)),
            pl.BlockSpec((_K, D), lambda i: (0, 0)),
        ],
        out_specs=pl.BlockSpec((1, 1, _M_BLK), lambda i: (i, 0, 0)),
        out_shape=jax.ShapeDtypeStruct((grid, 1, _M_BLK), jnp.int32),
    )(patches, codebook)
    return tokens.reshape(B, L // D)
